# Initial kernel scaffold; baseline (speedup 1.0000x reference)
import jax
import jax.numpy as jnp
from jax.experimental import pallas as pl


def kernel(feat_map, image, rpn_w, rpn_b, cls_w, cls_b, bbox_w, bbox_b):
    raise NotImplementedError("write your pallas kernel here")


# speedup vs baseline: 15.5449x; 15.5449x over previous
import functools

import numpy as np
import jax
import jax.numpy as jnp
from jax import lax
from jax.experimental import pallas as pl
from jax.experimental.pallas import tpu as pltpu

_SCALES = (128.0, 256.0, 512.0)
_RATIOS = (0.5, 1.0, 2.0)
_NUM_ANCHORS = 9
_PRE_NMS_TOPK = 10000
_POST_NMS_TOPK = 2000
_NMS_THRESH = 0.7
_MIN_SIZE = 16.0
_LOG_MAX = float(np.log(1000.0 / 16.0))

_N = 22500
_ROWS = 184
_LANES = 128
_NPAD = _ROWS * _LANES
_BIG = np.int32(1 << 30)


def _nms_body(cls_ref, d0, d1, d2, d3, a0, a1, a2, a3, out_ref,
              x1s, y1s, x2s, y2s):
    f32 = jnp.float32
    neg_inf = f32(-jnp.inf)

    probs = jax.nn.sigmoid(cls_ref[...])

    aw = a2[...] - a0[...]
    ah = a3[...] - a1[...]
    acx = a0[...] + 0.5 * aw
    acy = a1[...] + 0.5 * ah
    dw = jnp.minimum(d2[...], _LOG_MAX)
    dh = jnp.minimum(d3[...], _LOG_MAX)
    pcx = d0[...] * aw + acx
    pcy = d1[...] * ah + acy
    pw = jnp.exp(dw) * aw
    ph = jnp.exp(dh) * ah
    x1 = jnp.clip(pcx - 0.5 * pw, 0.0, 800.0)
    y1 = jnp.clip(pcy - 0.5 * ph, 0.0, 800.0)
    x2 = jnp.clip(pcx + 0.5 * pw, 0.0, 800.0)
    y2 = jnp.clip(pcy + 0.5 * ph, 0.0, 800.0)
    valid = ((x2 - x1) >= _MIN_SIZE) & ((y2 - y1) >= _MIN_SIZE)
    areas = (x2 - x1) * (y2 - y1)

    x1s[...] = x1
    y1s[...] = y1
    x2s[...] = x2
    y2s[...] = y2

    rid = lax.broadcasted_iota(jnp.int32, (_ROWS, _LANES), 0)
    lid = lax.broadcasted_iota(jnp.int32, (_ROWS, _LANES), 1)
    idv = rid * _LANES + lid

    key = lax.bitcast_convert_type(probs, jnp.int32)

    def _thresh_step(_, lohi):
        lo, hi = lohi
        mid = (lo + hi) // 2
        cnt = jnp.sum(jnp.where(key >= mid, 1, 0))
        big = cnt >= _PRE_NMS_TOPK
        return (jnp.where(big, mid, lo), jnp.where(big, hi, mid))

    lo, hi = lax.fori_loop(0, 31, _thresh_step, (jnp.int32(0), _BIG))
    kth = lo
    n_above = jnp.sum(jnp.where(key > kth, 1, 0))
    need = _PRE_NMS_TOPK - n_above
    ties = key == kth

    def _cut_step(_, lohi):
        lo, hi = lohi
        mid = (lo + hi) // 2
        cnt = jnp.sum(jnp.where(ties & (idv < mid), 1, 0))
        enough = cnt >= need
        return (jnp.where(enough, lo, mid), jnp.where(enough, mid, hi))

    lo2, hi2 = lax.fori_loop(0, 15, _cut_step,
                             (jnp.int32(0), jnp.int32(32768)))
    elig = (key > kth) | (ties & (idv < hi2))

    s0 = jnp.where(elig & valid, probs, neg_inf)

    pm = jnp.max(probs)
    tsel = probs == pm
    tidx = jnp.min(jnp.where(tsel, idv, _BIG))
    tfirst = tsel & (idv == tidx)
    tbx1 = jnp.max(jnp.where(tfirst, x1, neg_inf))
    tby1 = jnp.max(jnp.where(tfirst, y1, neg_inf))
    tbx2 = jnp.max(jnp.where(tfirst, x2, neg_inf))
    tby2 = jnp.max(jnp.where(tfirst, y2, neg_inf))
    tsc = jnp.max(jnp.where(tfirst & valid, probs, neg_inf))

    lane1 = lax.broadcasted_iota(jnp.int32, (1, _LANES), 1)

    def body(i, s):
        m = jnp.max(s)
        idx = jnp.min(jnp.where(s == m, idv, _BIG))
        r = lax.shift_right_logical(idx, 7)
        l = jnp.bitwise_and(idx, 127)
        lmask = lane1 == l
        bx1 = jnp.max(jnp.where(lmask, x1s[pl.ds(r, 1), :], neg_inf))
        by1 = jnp.max(jnp.where(lmask, y1s[pl.ds(r, 1), :], neg_inf))
        bx2 = jnp.max(jnp.where(lmask, x2s[pl.ds(r, 1), :], neg_inf))
        by2 = jnp.max(jnp.where(lmask, y2s[pl.ds(r, 1), :], neg_inf))
        barea = (bx2 - bx1) * (by2 - by1)

        live = m != neg_inf
        ox1 = jnp.where(live, bx1, tbx1)
        oy1 = jnp.where(live, by1, tby1)
        ox2 = jnp.where(live, bx2, tbx2)
        oy2 = jnp.where(live, by2, tby2)
        osc = jnp.where(live, m, tsc)
        rowv = jnp.where(lane1 == 0, ox1,
               jnp.where(lane1 == 1, oy1,
               jnp.where(lane1 == 2, ox2,
               jnp.where(lane1 == 3, oy2,
               jnp.where(lane1 == 4, osc, 0.0)))))
        out_ref[pl.ds(i, 1), :] = rowv

        xx1 = jnp.maximum(bx1, x1)
        yy1 = jnp.maximum(by1, y1)
        xx2 = jnp.minimum(bx2, x2)
        yy2 = jnp.minimum(by2, y2)
        inter = jnp.maximum(xx2 - xx1, 0.0) * jnp.maximum(yy2 - yy1, 0.0)
        iou = inter / (barea + areas - inter + 1e-9)
        return jnp.where(iou > _NMS_THRESH, neg_inf, s)

    lax.fori_loop(0, _POST_NMS_TOPK, body, s0)


@functools.partial(jax.jit, static_argnames=("interpret",))
def _run_nms(cls_p, d, a, interpret=False):
    return pl.pallas_call(
        _nms_body,
        out_shape=jax.ShapeDtypeStruct((_POST_NMS_TOPK, _LANES), jnp.float32),
        scratch_shapes=[pltpu.VMEM((_ROWS, _LANES), jnp.float32)] * 4,
        interpret=interpret,
    )(cls_p, *d, *a)


def _conv2d(x, w, b, padding):
    y = lax.conv_general_dilated(x, w, window_strides=(1, 1), padding=padding,
                                 dimension_numbers=('NCHW', 'OIHW', 'NCHW'))
    return y + b[None, :, None, None]


def _anchor_base(feat_h, feat_w, stride_h, stride_w):
    scales = jnp.asarray(_SCALES, jnp.float32)
    ratios = jnp.asarray(_RATIOS, jnp.float32)
    h_ratios = jnp.sqrt(ratios)
    w_ratios = 1.0 / h_ratios
    ws = (w_ratios[:, None] * scales[None, :]).reshape(-1)
    hs = (h_ratios[:, None] * scales[None, :]).reshape(-1)
    base = jnp.stack([-ws / 2.0, -hs / 2.0, ws / 2.0, hs / 2.0], axis=1)
    shift_x = (jnp.arange(feat_w, dtype=jnp.float32) + 0.5) * stride_w
    shift_y = (jnp.arange(feat_h, dtype=jnp.float32) + 0.5) * stride_h
    sy, sx = jnp.meshgrid(shift_y, shift_x, indexing='ij')
    shifts = jnp.stack([sx.reshape(-1), sy.reshape(-1),
                        sx.reshape(-1), sy.reshape(-1)], axis=1)
    return (shifts[:, None, :] + base[None, :, :]).reshape(-1, 4)


def _pad2d(v, fill):
    return jnp.pad(v, (0, _NPAD - _N), constant_values=fill).reshape(_ROWS, _LANES)


def kernel(feat_map, image, rpn_w, rpn_b, cls_w, cls_b, bbox_w, bbox_b,
           interpret=False):
    rpn_feat = jax.nn.relu(_conv2d(feat_map, rpn_w, rpn_b, 'SAME'))
    cls_scores = _conv2d(rpn_feat, cls_w, cls_b, 'VALID')
    box_pred = _conv2d(rpn_feat, bbox_w, bbox_b, 'VALID')
    B, _, H, W = box_pred.shape
    cls_flat = jnp.transpose(cls_scores, (0, 2, 3, 1)).reshape(-1)
    box_flat = jnp.transpose(box_pred.reshape(B, _NUM_ANCHORS, 4, H, W),
                             (0, 3, 4, 1, 2)).reshape(-1, 4)
    box_flat = lax.stop_gradient(box_flat)
    cls_flat = lax.stop_gradient(cls_flat)
    anchors = _anchor_base(H, W, image.shape[-2] // H, image.shape[-1] // W)

    cls_p = _pad2d(cls_flat, -1e30)
    d = [_pad2d(box_flat[:, i], 0.0) for i in range(4)]
    a = [_pad2d(anchors[:, i], 0.0) for i in range(4)]

    outp = _run_nms(cls_p, tuple(d), tuple(a), interpret=interpret)
    return outp[:, :4], outp[:, 4]
